# baseline (speedup 1.0000x reference)
import functools

import jax
import jax.numpy as jnp
from jax.experimental import pallas as pl
from jax.experimental.pallas import tpu as pltpu

N = 10000
E = 160000
D = 256
_RB = 2000


def _pool_body(x_ref, w_ref, b_ref, o_ref):
    acc = jax.lax.dot_general(
        x_ref[...], w_ref[...], (((1,), (1,)), ((), ())),
        preferred_element_type=jnp.float32)
    o_ref[...] = jnp.maximum(acc + b_ref[...], 0.0)


def _pool_matmul(x, W, b):
    return pl.pallas_call(
        _pool_body,
        grid=(N // _RB,),
        in_specs=[
            pl.BlockSpec((_RB, D), lambda i: (i, 0)),
            pl.BlockSpec((D, D), lambda i: (0, 0)),
            pl.BlockSpec((1, D), lambda i: (0, 0)),
        ],
        out_specs=pl.BlockSpec((_RB, D), lambda i: (i, 0)),
        out_shape=jax.ShapeDtypeStruct((N, D), jnp.float32),
    )(x, W, b.reshape(1, D))


def _out_body(x_ref, ws_ref, e_ref, we_ref, b_ref, o_ref, *, relu):
    acc = jax.lax.dot_general(
        x_ref[...], ws_ref[...], (((1,), (1,)), ((), ())),
        preferred_element_type=jnp.float32)
    acc = acc + jax.lax.dot_general(
        e_ref[...], we_ref[...], (((1,), (1,)), ((), ())),
        preferred_element_type=jnp.float32)
    acc = acc + b_ref[...]
    if relu:
        acc = jnp.maximum(acc, 0.0)
    o_ref[...] = acc


def _out_matmul(x, Wself, eve, Weve, b, relu):
    return pl.pallas_call(
        functools.partial(_out_body, relu=relu),
        grid=(N // _RB,),
        in_specs=[
            pl.BlockSpec((_RB, D), lambda i: (i, 0)),
            pl.BlockSpec((D, D), lambda i: (0, 0)),
            pl.BlockSpec((_RB, D), lambda i: (i, 0)),
            pl.BlockSpec((D, D), lambda i: (0, 0)),
            pl.BlockSpec((1, D), lambda i: (0, 0)),
        ],
        out_specs=pl.BlockSpec((_RB, D), lambda i: (i, 0)),
        out_shape=jax.ShapeDtypeStruct((N, D), jnp.float32),
    )(x, Wself, eve, Weve, b.reshape(1, D))


def _segment_eve_xla(h, src, dst, dww, dwb):
    m = jnp.take(h, src, axis=0)
    xmax = jax.ops.segment_max(m, dst, num_segments=N)
    xmin = jax.ops.segment_min(m, dst, num_segments=N)
    xmax = jnp.where(jnp.isfinite(xmax), xmax, 0.0)
    xmin = jnp.where(jnp.isfinite(xmin), xmin, 0.0)
    return jax.nn.relu(dww[0] * xmax + dww[1] * xmin + dwb[0])


def _layer(x, src, dst, Wpool, bpool, dww, dwb, Weve, Wself, bias, relu):
    h = _pool_matmul(x, Wpool, bpool)
    eve = _segment_eve_xla(h, src, dst, dww, dwb)
    return _out_matmul(x, Wself, eve, Weve, bias, relu)


def kernel(x, edge_index, c1_Wpool, c1_bpool, c1_dww, c1_dwb, c1_Weve, c1_Wself, c1_bias, c2_Wpool, c2_bpool, c2_dww, c2_dwb, c2_Weve, c2_Wself, c2_bias):
    src = edge_index[0]
    dst = edge_index[1]
    h = _layer(x, src, dst, c1_Wpool, c1_bpool, c1_dww, c1_dwb, c1_Weve,
               c1_Wself, c1_bias, relu=True)
    return _layer(h, src, dst, c2_Wpool, c2_bpool, c2_dww, c2_dwb, c2_Weve,
                  c2_Wself, c2_bias, relu=False)
